# Initial kernel scaffold; baseline (speedup 1.0000x reference)
import jax
import jax.numpy as jnp
from jax.experimental import pallas as pl


def kernel(feats, labels, training):
    raise NotImplementedError("write your pallas kernel here")


# speedup vs baseline: 1.1860x; 1.1860x over previous
import jax
import jax.numpy as jnp
from jax import lax
from jax.experimental import pallas as pl
from jax.experimental.pallas import tpu as pltpu
from jax.experimental.pallas import tpu_sc as plsc

N_TOTAL = 8192
C = 512
N_GROUPS = 16
GROUP_N = N_TOTAL // N_GROUPS

_NUM_WORKERS = 32
_ROWS_PER_WORKER = N_TOTAL // _NUM_WORKERS
_CHUNK = 128


def _score_perm_body(x_ref, idx_ref):
    g = pl.program_id(0)
    x = x_ref[...]

    n2 = jnp.sum(x * x, axis=1, keepdims=True)
    norm = jnp.maximum(jnp.sqrt(n2), 1e-12)
    y = x / norm

    sim = lax.dot_general(
        y, y, dimension_numbers=(((1,), (1,)), ((), ())),
        preferred_element_type=jnp.float32,
    )
    scores_col = jnp.sum(sim, axis=1, keepdims=True) / GROUP_N

    n_ids = lax.broadcasted_iota(jnp.int32, (GROUP_N, GROUP_N), 0)
    m_ids = lax.broadcasted_iota(jnp.int32, (GROUP_N, GROUP_N), 1)
    eye = (n_ids == m_ids).astype(jnp.float32)
    scores_row = lax.dot_general(
        scores_col, eye, dimension_numbers=(((0,), (0,)), ((), ())),
        preferred_element_type=jnp.float32,
    )

    gt = scores_row > scores_col
    eq = (scores_row == scores_col) & (m_ids < n_ids)
    rank = jnp.sum((gt | eq).astype(jnp.int32), axis=1, keepdims=True)

    onehot = rank == m_ids
    perm = jnp.sum(jnp.where(onehot, n_ids, 0), axis=0, keepdims=True)

    idx_ref[...] = (perm + g * GROUP_N).reshape(1, 1, GROUP_N)


def _sorted_indices(feats):
    return pl.pallas_call(
        _score_perm_body,
        grid=(N_GROUPS,),
        in_specs=[pl.BlockSpec((GROUP_N, C), lambda g: (g, 0))],
        out_specs=pl.BlockSpec((1, 1, GROUP_N), lambda g: (g, 0, 0)),
        out_shape=jax.ShapeDtypeStruct((N_GROUPS, 1, GROUP_N), jnp.int32),
    )(feats)


def _gather_body(feats_hbm, idx_hbm, out_hbm, idx_v, rows_v, sem):
    wid = lax.axis_index("s") * 2 + lax.axis_index("c")
    for chunk in range(_ROWS_PER_WORKER // _CHUNK):
        base = wid * _ROWS_PER_WORKER + chunk * _CHUNK
        pltpu.sync_copy(idx_hbm.at[pl.ds(base, _CHUNK)], idx_v)
        pltpu.async_copy(feats_hbm.at[idx_v], rows_v, sem).wait()
        pltpu.sync_copy(rows_v, out_hbm.at[pl.ds(base, _CHUNK)])


def _gather_rows(feats, idx):
    gather = pl.kernel(
        _gather_body,
        out_type=jax.ShapeDtypeStruct((N_TOTAL, C), jnp.float32),
        scratch_types=[
            pltpu.VMEM((_CHUNK,), jnp.int32),
            pltpu.VMEM((_CHUNK, C), jnp.float32),
            pltpu.SemaphoreType.DMA,
        ],
        mesh=plsc.VectorSubcoreMesh(core_axis_name="c", subcore_axis_name="s"),
    )
    return gather(feats, idx)


def kernel(feats, labels, training):
    del labels, training
    idx = _sorted_indices(feats).reshape(N_TOTAL)
    sorted_rows = _gather_rows(feats, idx)
    out_sorted = sorted_rows.reshape(N_GROUPS, GROUP_N * C)
    out_input = feats.reshape(N_GROUPS, GROUP_N * C)
    return (out_sorted, out_input)
